# baseline (speedup 1.0000x reference)
import dataclasses
import functools

import jax
import jax.numpy as jnp
from jax import lax
from jax.experimental import pallas as pl
from jax.experimental.pallas import tpu as pltpu
from jax.experimental.pallas import tpu_sc as plsc

_ROWS = 16384
_C = 2048
_L = 16
_RT = 8
_NC = 2
_NW = 32
_NBI = 2
_NBO = 4
_GRP = 4


def kernel(z, perm):
    perm32 = perm.astype(jnp.int32)
    mesh = plsc.VectorSubcoreMesh(
        core_axis_name="core", subcore_axis_name="subcore"
    )

    cp = pltpu.CompilerParams()
    if "needs_layout_passes" in pltpu.CompilerParams.__dataclass_fields__:
        cp = dataclasses.replace(cp, needs_layout_passes=False)

    @functools.partial(
        pl.kernel,
        out_type=jax.ShapeDtypeStruct((_ROWS, _C), jnp.float32),
        mesh=mesh,
        compiler_params=cp,
        scratch_types=[
            pltpu.VMEM((_C,), jnp.int32),
            pltpu.VMEM((_NBI, _RT, _C), jnp.float32),
            pltpu.VMEM((_NBO, _RT, _C), jnp.float32),
            pltpu.SemaphoreType.DMA((_NBI,)),
            pltpu.SemaphoreType.DMA((_NBO,)),
            pltpu.SemaphoreType.DMA,
        ],
    )
    def run(z_hbm, perm_hbm, out_hbm, perm_v, inb, outb, isem, osem, psem):
        cid = lax.axis_index("core")
        sid = lax.axis_index("subcore")
        wid = sid * _NC + cid
        rows_pw = _ROWS // _NW
        steps = rows_pw // _RT
        base = wid * rows_pw

        pltpu.async_copy(perm_hbm, perm_v, psem).wait()

        def in_slice(s):
            return z_hbm.at[pl.ds(base + s * _RT, _RT)]

        def out_slice(s):
            return out_hbm.at[pl.ds(base + s * _RT, _RT)]

        for b in range(_NBI):
            pltpu.async_copy(in_slice(b), inb.at[b], isem.at[b])

        @pl.loop(0, steps // _GRP)
        def _(g):
            for k in range(_GRP):
                s = g * _GRP + k
                bi = k % _NBI
                bo = k % _NBO
                pltpu.make_async_copy(in_slice(s), inb.at[bi], isem.at[bi]).wait()

                @pl.when(s >= _NBO)
                def _():
                    pltpu.make_async_copy(
                        outb.at[bo], out_slice(s - _NBO), osem.at[bo]
                    ).wait()

                i_ref = inb.at[bi]
                o_ref = outb.at[bo]

                @plsc.parallel_loop(0, _C // _L, unroll=8)
                def _(cb):
                    col = perm_v[pl.ds(cb * _L, _L)]
                    for r in range(_RT):
                        rowidx = jnp.full((_L,), r, jnp.int32)
                        o_ref[r, pl.ds(cb * _L, _L)] = plsc.load_gather(
                            i_ref, [rowidx, col]
                        )

                pltpu.async_copy(o_ref, out_slice(s), osem.at[bo])

                @pl.when(s + _NBI < steps)
                def _():
                    pltpu.async_copy(in_slice(s + _NBI), inb.at[bi], isem.at[bi])

        for k in range(_NBO):
            s = steps - _NBO + k
            pltpu.make_async_copy(
                outb.at[s % _NBO], out_slice(s), osem.at[s % _NBO]
            ).wait()

    z_out = run(z, perm32)
    return (z_out, jnp.zeros((), z.dtype))


# speedup vs baseline: 1.0282x; 1.0282x over previous
import dataclasses
import functools

import jax
import jax.numpy as jnp
from jax import lax
from jax.experimental import pallas as pl
from jax.experimental.pallas import tpu as pltpu
from jax.experimental.pallas import tpu_sc as plsc

_ROWS = 16384
_C = 2048
_L = 16
_RT = 8
_NC = 2
_NW = 32
_NBI = 4
_NBO = 2
_GRP = 4


def kernel(z, perm):
    perm32 = perm.astype(jnp.int32)
    mesh = plsc.VectorSubcoreMesh(
        core_axis_name="core", subcore_axis_name="subcore"
    )

    cp = pltpu.CompilerParams()
    if "needs_layout_passes" in pltpu.CompilerParams.__dataclass_fields__:
        cp = dataclasses.replace(cp, needs_layout_passes=False)

    @functools.partial(
        pl.kernel,
        out_type=jax.ShapeDtypeStruct((_ROWS, _C), jnp.float32),
        mesh=mesh,
        compiler_params=cp,
        scratch_types=[
            pltpu.VMEM((_C,), jnp.int32),
            pltpu.VMEM((_NBI, _RT, _C), jnp.float32),
            pltpu.VMEM((_NBO, _RT, _C), jnp.float32),
            pltpu.SemaphoreType.DMA((_NBI,)),
            pltpu.SemaphoreType.DMA((_NBO,)),
            pltpu.SemaphoreType.DMA,
        ],
    )
    def run(z_hbm, perm_hbm, out_hbm, perm_v, inb, outb, isem, osem, psem):
        cid = lax.axis_index("core")
        sid = lax.axis_index("subcore")
        wid = sid * _NC + cid
        rows_pw = _ROWS // _NW
        steps = rows_pw // _RT
        base = wid * rows_pw

        pltpu.async_copy(perm_hbm, perm_v, psem).wait()

        def in_slice(s):
            return z_hbm.at[pl.ds(base + s * _RT, _RT)]

        def out_slice(s):
            return out_hbm.at[pl.ds(base + s * _RT, _RT)]

        for b in range(_NBI):
            pltpu.async_copy(in_slice(b), inb.at[b], isem.at[b])

        @pl.loop(0, steps // _GRP)
        def _(g):
            for k in range(_GRP):
                s = g * _GRP + k
                bi = k % _NBI
                bo = k % _NBO
                pltpu.make_async_copy(in_slice(s), inb.at[bi], isem.at[bi]).wait()

                @pl.when(s >= _NBO)
                def _():
                    pltpu.make_async_copy(
                        outb.at[bo], out_slice(s - _NBO), osem.at[bo]
                    ).wait()

                i_ref = inb.at[bi]
                o_ref = outb.at[bo]

                @plsc.parallel_loop(0, _C // _L, unroll=8)
                def _(cb):
                    col = perm_v[pl.ds(cb * _L, _L)]
                    for r in range(_RT):
                        rowidx = jnp.full((_L,), r, jnp.int32)
                        o_ref[r, pl.ds(cb * _L, _L)] = plsc.load_gather(
                            i_ref, [rowidx, col]
                        )

                pltpu.async_copy(o_ref, out_slice(s), osem.at[bo])

                @pl.when(s + _NBI < steps)
                def _():
                    pltpu.async_copy(in_slice(s + _NBI), inb.at[bi], isem.at[bi])

        for k in range(_NBO):
            s = steps - _NBO + k
            pltpu.make_async_copy(
                outb.at[s % _NBO], out_slice(s), osem.at[s % _NBO]
            ).wait()

    z_out = run(z, perm32)
    return (z_out, jnp.zeros((), z.dtype))


# speedup vs baseline: 1.0366x; 1.0081x over previous
import dataclasses
import functools

import jax
import jax.numpy as jnp
from jax.experimental import pallas as pl
from jax.experimental.pallas import tpu as pltpu
from jax.experimental.pallas import tpu_sc as plsc

_ROWS = 16384
_C = 2048
_L = 16
_RT = 8


def kernel(z, perm):
    perm32 = perm.astype(jnp.int32)
    mesh = plsc.VectorSubcoreMesh(
        core_axis_name="core", subcore_axis_name="subcore"
    )

    cp = pltpu.CompilerParams()
    if "needs_layout_passes" in pltpu.CompilerParams.__dataclass_fields__:
        cp = dataclasses.replace(cp, needs_layout_passes=False)

    @functools.partial(
        pl.kernel,
        out_type=jax.ShapeDtypeStruct((_ROWS, _C), jnp.float32),
        mesh=mesh,
        compiler_params=cp,
        scratch_types=[
            pltpu.VMEM((_C,), jnp.int32),
            pltpu.SemaphoreType.DMA,
        ],
    )
    def run(z_hbm, perm_hbm, out_hbm, perm_v, sem):
        pltpu.async_copy(perm_hbm, perm_v, sem).wait()

        def tile_body(z_vmem, o_vmem):
            @plsc.parallel_loop(0, _C // _L, unroll=8)
            def _(cb):
                col = perm_v[pl.ds(cb * _L, _L)]
                for r in range(_RT):
                    rowidx = jnp.full((_L,), r, jnp.int32)
                    o_vmem[r, pl.ds(cb * _L, _L)] = plsc.load_gather(
                        z_vmem, [rowidx, col]
                    )

        pltpu.emit_pipeline(
            tile_body,
            grid=(_ROWS // _RT,),
            in_specs=[
                pl.BlockSpec(
                    (_RT, _C),
                    lambda i: (i, 0),
                    pipeline_mode=pl.Buffered(buffer_count=4, use_lookahead=True),
                )
            ],
            out_specs=[pl.BlockSpec((_RT, _C), lambda i: (i, 0))],
            core_axis_name=("core", "subcore"),
            dimension_semantics=(pltpu.PARALLEL,),
        )(z_hbm, out_hbm)

    z_out = run(z, perm32)
    return (z_out, jnp.zeros((), z.dtype))
